# baseline (speedup 1.0000x reference)
import functools

import jax
import jax.numpy as jnp
from jax import lax
from jax.experimental import pallas as pl
from jax.experimental.pallas import tpu as pltpu
from jax.experimental.pallas import tpu_sc as plsc

B, Q, T, H, W = 4, 128, 64, 64, 64
HW = H * W
WIN = 7
NWIN = WIN * WIN
WPAD = 64
HALF = WIN // 2
C_OCC = 8
NO_ELECTRON_WEIGHT = 0.1
LOG_2PI = 1.8378770664093453

NC, NS = 2, 16
NWORK = NC * NS
PAIRS = B * T
PPW = PAIRS // NWORK


def _bce(x, y):
    return jnp.maximum(x, 0.0) - x * y + jnp.log1p(jnp.exp(-jnp.abs(x)))


def _sc_window_gather(bin_hbm, true_hbm, incr_hbm, incc_hbm, match_hbm,
                      boff_hbm, toff_hbm, out_bin, out_true,
                      incr_v, incc_v, m_v, boff_v, toff_v,
                      bidx_v, tidx_v, gbin_v, gtrue_v, sem_b, sem_t):
    wid = lax.axis_index("s") * NC + lax.axis_index("c")
    base = wid * PPW

    pltpu.sync_copy(incr_hbm.at[pl.ds(base, PPW)], incr_v.at[pl.ds(0, PPW)])
    pltpu.sync_copy(incc_hbm.at[pl.ds(base, PPW)], incc_v.at[pl.ds(0, PPW)])
    pltpu.sync_copy(match_hbm.at[pl.ds(base, PPW)], m_v.at[pl.ds(0, PPW)])
    pltpu.sync_copy(boff_hbm, boff_v)
    pltpu.sync_copy(toff_hbm, toff_v)

    lanes = lax.iota(jnp.int32, 16)
    r_vec = incr_v[...].astype(jnp.int32)
    c_vec = incc_v[...].astype(jnp.int32)
    m_vec = m_v[...]
    pair = base + lanes
    b_vec = lax.shift_right_logical(pair, 6)
    t_vec = pair & (T - 1)
    av_bin = (b_vec * (H * W * Q) + (r_vec - HALF) * (W * Q)
              + (c_vec - HALF) * Q + m_vec)
    av_true = (b_vec * (H * W * T) + (r_vec - HALF) * (W * T)
               + (c_vec - HALF) * T + t_vec)

    for j in range(PPW):
        jb = jnp.full((16,), j, jnp.int32)
        a_bin = av_bin.at[jb].get(mode='promise_in_bounds')
        a_true = av_true.at[jb].get(mode='promise_in_bounds')
        for v in range(4):
            off_b = boff_v[pl.ds(v * 16, 16)]
            off_t = toff_v[pl.ds(v * 16, 16)]
            bidx_v[j, pl.ds(v * 16, 16)] = a_bin + off_b
            tidx_v[j, pl.ds(v * 16, 16)] = a_true + off_t

    copies = []
    for j in range(PPW):
        copies.append(pltpu.async_copy(bin_hbm.at[bidx_v.at[j]], gbin_v.at[j], sem_b))
        copies.append(pltpu.async_copy(true_hbm.at[tidx_v.at[j]], gtrue_v.at[j], sem_t))
    for c in copies:
        c.wait()

    pltpu.sync_copy(gbin_v, out_bin.at[pl.ds(wid * PPW, PPW)])
    pltpu.sync_copy(gtrue_v, out_true.at[pl.ds(wid * PPW, PPW)])


_sc_gather_call = pl.kernel(
    _sc_window_gather,
    mesh=plsc.VectorSubcoreMesh(core_axis_name="c", subcore_axis_name="s"),
    out_type=[jax.ShapeDtypeStruct((PAIRS, WPAD), jnp.float32),
              jax.ShapeDtypeStruct((PAIRS, WPAD), jnp.float32)],
    scratch_types=[
        pltpu.VMEM((16,), jnp.float32),
        pltpu.VMEM((16,), jnp.float32),
        pltpu.VMEM((16,), jnp.int32),
        pltpu.VMEM((WPAD,), jnp.int32),
        pltpu.VMEM((WPAD,), jnp.int32),
        pltpu.VMEM((PPW, WPAD), jnp.int32),
        pltpu.VMEM((PPW, WPAD), jnp.int32),
        pltpu.VMEM((PPW, WPAD), jnp.float32),
        pltpu.VMEM((PPW, WPAD), jnp.float32),
        pltpu.SemaphoreType.DMA,
        pltpu.SemaphoreType.DMA,
    ])


def _main_kernel(portion_ref, true_ref, matched_ref, inc_ref,
                 ie_ref, packed_ref, occ_ref, occ_oh_ref, out_ref, acc_ref):
    b = pl.program_id(0)

    matched = matched_ref[0]
    q_iota = lax.broadcasted_iota(jnp.int32, (Q, T), 0)
    onehot = (q_iota == matched).astype(jnp.float32)

    true_b = true_ref[0]

    rp = lax.dot_general(
        portion_ref[0], onehot, (((1,), (0,)), ((), ())),
        precision=lax.Precision.HIGHEST,
        preferred_element_type=jnp.float32)
    p = jax.nn.sigmoid(rp)
    num_t = 2.0 * jnp.sum(p * true_b, axis=0, keepdims=True)
    den_t = jnp.sum(p, axis=0, keepdims=True) + jnp.sum(true_b, axis=0, keepdims=True)
    dice_b = jnp.sum(1.0 - (num_t + 1.0) / (den_t + 1.0))

    labels = jnp.max(onehot, axis=1, keepdims=True)
    wts = jnp.where(labels > 0.0, 1.0, NO_ELECTRON_WEIGHT)
    x_ie = ie_ref[0].reshape(Q, 1)
    class_b = jnp.sum(wts * _bce(x_ie, labels))

    g = lax.dot_general(
        onehot, packed_ref[0], (((0,), (0,)), ((), ())),
        precision=lax.Precision.HIGHEST,
        preferred_element_type=jnp.float32)
    ix = inc_ref[0, 0:1, :].reshape(T, 1)
    iy = inc_ref[0, 1:2, :].reshape(T, 1)
    d0 = ix - g[:, 0:1]
    d1 = iy - g[:, 1:2]
    l00 = g[:, 2:3]
    l10 = g[:, 3:4]
    l11 = g[:, 4:5]
    z0 = d0 / l00
    z1 = (d1 - l10 * z0) / l11
    nll_b = jnp.sum(0.5 * (z0 * z0 + z1 * z1)
                    + jnp.log(jnp.abs(l00)) + jnp.log(jnp.abs(l11)) + LOG_2PI)

    @pl.when(b == 0)
    def _init():
        for i in range(4):
            acc_ref[i] = 0.0

    acc_ref[0] = acc_ref[0] + class_b
    acc_ref[1] = acc_ref[1] + dice_b
    acc_ref[2] = acc_ref[2] + nll_b

    @pl.when(b == B - 1)
    def _final():
        xo = occ_ref[:, :]
        m = jnp.max(xo, axis=1, keepdims=True)
        lse = m + jnp.log(jnp.sum(jnp.exp(xo - m), axis=1, keepdims=True))
        occ_loss = -jnp.sum(occ_oh_ref[:, :] * (xo - lse)) / B
        out_ref[0] = (acc_ref[0] / (B * Q)
                      + acc_ref[1] / (B * T)
                      + acc_ref[2] / (B * T)
                      + occ_loss)


def _combine_kernel(gbin_ref, gtrue_ref, partial_ref, out_ref):
    k_iota = lax.broadcasted_iota(jnp.int32, (PAIRS, WPAD), 1)
    valid = (k_iota < NWIN).astype(jnp.float32)
    bce_sum = jnp.sum(valid * _bce(gbin_ref[:, :], gtrue_ref[:, :]))
    out_ref[0] = partial_ref[0] + bce_sum / (B * T * NWIN)


@jax.jit
def kernel(is_electron_logit, positions, position_std_dev_cholesky, true_segmap,
           binary_mask_logits, portion_logits, occupancy_logits, incidence_points,
           matched_pred, occupancy_target):
    portion = portion_logits.reshape(B, HW, Q)
    true = true_segmap.reshape(B, HW, T)
    matched3 = matched_pred.reshape(B, 1, T)
    inc_t = incidence_points.transpose(0, 2, 1)
    ie = is_electron_logit.reshape(B, 1, Q)
    pos = positions.reshape(B, Q, 2)
    chol = position_std_dev_cholesky.reshape(B, Q, 2, 2)
    packed = jnp.concatenate(
        [pos, chol[..., 0, 0:1], chol[..., 1, 0:1], chol[..., 1, 1:2],
         jnp.zeros((B, Q, 3), jnp.float32)], axis=-1)
    occ_oh = (occupancy_target[:, None] ==
              jnp.arange(C_OCC, dtype=jnp.int32)[None, :]).astype(jnp.float32)

    k = jnp.minimum(jnp.arange(WPAD, dtype=jnp.int32), NWIN - 1)
    boff = (k // WIN) * (W * Q) + (k % WIN) * Q
    toff = (k // WIN) * (W * T) + (k % WIN) * T

    inc_r = incidence_points[..., 0].reshape(-1)
    inc_c = incidence_points[..., 1].reshape(-1)
    gbin, gtrue = _sc_gather_call(
        binary_mask_logits.reshape(-1), true_segmap.reshape(-1),
        inc_r, inc_c, matched_pred.reshape(-1), boff, toff)

    partial = pl.pallas_call(
        _main_kernel,
        grid=(B,),
        in_specs=[
            pl.BlockSpec((1, HW, Q), lambda b: (b, 0, 0)),
            pl.BlockSpec((1, HW, T), lambda b: (b, 0, 0)),
            pl.BlockSpec((1, 1, T), lambda b: (b, 0, 0)),
            pl.BlockSpec((1, 2, T), lambda b: (b, 0, 0)),
            pl.BlockSpec((1, 1, Q), lambda b: (b, 0, 0)),
            pl.BlockSpec((1, Q, 8), lambda b: (b, 0, 0)),
            pl.BlockSpec((B, C_OCC), lambda b: (0, 0)),
            pl.BlockSpec((B, C_OCC), lambda b: (0, 0)),
        ],
        out_specs=pl.BlockSpec(memory_space=pltpu.SMEM),
        out_shape=jax.ShapeDtypeStruct((1,), jnp.float32),
        scratch_shapes=[pltpu.SMEM((8,), jnp.float32)],
    )(portion, true, matched3, inc_t, ie, packed, occupancy_logits, occ_oh)

    out = pl.pallas_call(
        _combine_kernel,
        in_specs=[
            pl.BlockSpec((PAIRS, WPAD), lambda: (0, 0)),
            pl.BlockSpec((PAIRS, WPAD), lambda: (0, 0)),
            pl.BlockSpec(memory_space=pltpu.SMEM),
        ],
        out_specs=pl.BlockSpec(memory_space=pltpu.SMEM),
        out_shape=jax.ShapeDtypeStruct((1,), jnp.float32),
    )(gbin, gtrue, partial)
    return out[0]


# speedup vs baseline: 1.0167x; 1.0167x over previous
import functools

import jax
import jax.numpy as jnp
from jax import lax
from jax.experimental import pallas as pl
from jax.experimental.pallas import tpu as pltpu
from jax.experimental.pallas import tpu_sc as plsc

B, Q, T, H, W = 4, 128, 64, 64, 64
HW = H * W
WIN = 7
NWIN = WIN * WIN
WPAD = 64
HALF = WIN // 2
C_OCC = 8
NO_ELECTRON_WEIGHT = 0.1
LOG_2PI = 1.8378770664093453

NC, NS = 2, 16
NWORK = NC * NS
PAIRS = B * T
PPW = PAIRS // NWORK


def _bce(x, y):
    return jnp.maximum(x, 0.0) - x * y + jnp.log1p(jnp.exp(-jnp.abs(x)))


NROW = PPW // 2


def _sc_window_gather(bin_hbm, true_hbm, inc_hbm, match_hbm,
                      boff_hbm, toff_hbm, out_bin, out_true,
                      incr_v, incc_v, m_v, boff_v, toff_v,
                      bidx_v, tidx_v, gbin_v, gtrue_v, sem_b, sem_t):
    wid = lax.axis_index("s") * NC + lax.axis_index("c")
    base = wid * PPW
    b_id = lax.shift_right_logical(wid, 3)

    pltpu.sync_copy(inc_hbm.at[pl.ds(base + b_id * T, PPW)],
                    incr_v.at[pl.ds(0, PPW)])
    pltpu.sync_copy(inc_hbm.at[pl.ds(base + b_id * T + T, PPW)],
                    incc_v.at[pl.ds(0, PPW)])
    pltpu.sync_copy(match_hbm.at[pl.ds(base, PPW)], m_v.at[pl.ds(0, PPW)])
    pltpu.sync_copy(boff_hbm, boff_v)
    pltpu.sync_copy(toff_hbm, toff_v)

    lanes = lax.iota(jnp.int32, 16)
    r_vec = incr_v[...].astype(jnp.int32)
    c_vec = incc_v[...].astype(jnp.int32)
    m_vec = m_v[...]
    t_vec = (base + lanes) & (T - 1)
    av_bin = (b_id * (H * W * Q) + (r_vec - HALF) * (W * Q)
              + (c_vec - HALF) * Q + m_vec)
    av_true = (b_id * (H * W * T) + (r_vec - HALF) * (W * T)
               + (c_vec - HALF) * T + t_vec)

    for j in range(PPW):
        jb = jnp.full((16,), j, jnp.int32)
        a_bin = av_bin.at[jb].get(mode='promise_in_bounds')
        a_true = av_true.at[jb].get(mode='promise_in_bounds')
        row, col = j // 2, (j % 2) * WPAD
        for v in range(4):
            off_b = boff_v[pl.ds(v * 16, 16)]
            off_t = toff_v[pl.ds(v * 16, 16)]
            bidx_v[row, pl.ds(col + v * 16, 16)] = a_bin + off_b
            tidx_v[row, pl.ds(col + v * 16, 16)] = a_true + off_t

    copies = []
    for j in range(NROW):
        copies.append(pltpu.async_copy(bin_hbm.at[bidx_v.at[j]], gbin_v.at[j], sem_b))
        copies.append(pltpu.async_copy(true_hbm.at[tidx_v.at[j]], gtrue_v.at[j], sem_t))
    for c in copies:
        c.wait()

    pltpu.sync_copy(gbin_v, out_bin.at[pl.ds(wid * NROW, NROW)])
    pltpu.sync_copy(gtrue_v, out_true.at[pl.ds(wid * NROW, NROW)])


_sc_gather_call = pl.kernel(
    _sc_window_gather,
    mesh=plsc.VectorSubcoreMesh(core_axis_name="c", subcore_axis_name="s"),
    out_type=[jax.ShapeDtypeStruct((PAIRS // 2, 2 * WPAD), jnp.float32),
              jax.ShapeDtypeStruct((PAIRS // 2, 2 * WPAD), jnp.float32)],
    scratch_types=[
        pltpu.VMEM((16,), jnp.float32),
        pltpu.VMEM((16,), jnp.float32),
        pltpu.VMEM((16,), jnp.int32),
        pltpu.VMEM((WPAD,), jnp.int32),
        pltpu.VMEM((WPAD,), jnp.int32),
        pltpu.VMEM((NROW, 2 * WPAD), jnp.int32),
        pltpu.VMEM((NROW, 2 * WPAD), jnp.int32),
        pltpu.VMEM((NROW, 2 * WPAD), jnp.float32),
        pltpu.VMEM((NROW, 2 * WPAD), jnp.float32),
        pltpu.SemaphoreType.DMA,
        pltpu.SemaphoreType.DMA,
    ])


def _main_kernel(portion_ref, true_ref, matched_ref, inc_ref,
                 ie_ref, packed_ref, occ_ref, occ_oh_ref, out_ref, acc_ref):
    b = pl.program_id(0)

    matched = matched_ref[0]
    q_iota = lax.broadcasted_iota(jnp.int32, (Q, T), 0)
    onehot = (q_iota == matched).astype(jnp.float32)

    true_b = true_ref[0]

    rp = lax.dot_general(
        portion_ref[0], onehot, (((1,), (0,)), ((), ())),
        precision=lax.Precision.HIGHEST,
        preferred_element_type=jnp.float32)
    p = jax.nn.sigmoid(rp)
    num_t = 2.0 * jnp.sum(p * true_b, axis=0, keepdims=True)
    den_t = jnp.sum(p, axis=0, keepdims=True) + jnp.sum(true_b, axis=0, keepdims=True)
    dice_b = jnp.sum(1.0 - (num_t + 1.0) / (den_t + 1.0))

    labels = jnp.max(onehot, axis=1, keepdims=True)
    wts = jnp.where(labels > 0.0, 1.0, NO_ELECTRON_WEIGHT)
    x_ie = ie_ref[0].reshape(Q, 1)
    class_b = jnp.sum(wts * _bce(x_ie, labels))

    g = lax.dot_general(
        onehot, packed_ref[0], (((0,), (0,)), ((), ())),
        precision=lax.Precision.HIGHEST,
        preferred_element_type=jnp.float32)
    ix = inc_ref[0, 0:1, :].reshape(T, 1)
    iy = inc_ref[0, 1:2, :].reshape(T, 1)
    d0 = ix - g[:, 0:1]
    d1 = iy - g[:, 1:2]
    l00 = g[:, 2:3]
    l10 = g[:, 3:4]
    l11 = g[:, 4:5]
    z0 = d0 / l00
    z1 = (d1 - l10 * z0) / l11
    nll_b = jnp.sum(0.5 * (z0 * z0 + z1 * z1)
                    + jnp.log(jnp.abs(l00)) + jnp.log(jnp.abs(l11)) + LOG_2PI)

    @pl.when(b == 0)
    def _init():
        for i in range(4):
            acc_ref[i] = 0.0

    acc_ref[0] = acc_ref[0] + class_b
    acc_ref[1] = acc_ref[1] + dice_b
    acc_ref[2] = acc_ref[2] + nll_b

    @pl.when(b == B - 1)
    def _final():
        xo = occ_ref[:, :]
        m = jnp.max(xo, axis=1, keepdims=True)
        lse = m + jnp.log(jnp.sum(jnp.exp(xo - m), axis=1, keepdims=True))
        occ_loss = -jnp.sum(occ_oh_ref[:, :] * (xo - lse)) / B
        out_ref[0] = (acc_ref[0] / (B * Q)
                      + acc_ref[1] / (B * T)
                      + acc_ref[2] / (B * T)
                      + occ_loss)


def _combine_kernel(gbin_ref, gtrue_ref, partial_ref, out_ref):
    k_iota = lax.broadcasted_iota(jnp.int32, (PAIRS // 2, 2 * WPAD), 1)
    valid = ((k_iota & (WPAD - 1)) < NWIN).astype(jnp.float32)
    bce_sum = jnp.sum(valid * _bce(gbin_ref[:, :], gtrue_ref[:, :]))
    out_ref[0] = partial_ref[0] + bce_sum / (B * T * NWIN)


@jax.jit
def kernel(is_electron_logit, positions, position_std_dev_cholesky, true_segmap,
           binary_mask_logits, portion_logits, occupancy_logits, incidence_points,
           matched_pred, occupancy_target):
    portion = portion_logits.reshape(B, HW, Q)
    true = true_segmap.reshape(B, HW, T)
    matched3 = matched_pred.reshape(B, 1, T)
    inc_t = incidence_points.transpose(0, 2, 1)
    ie = is_electron_logit.reshape(B, 1, Q)
    pos = positions.reshape(B, Q, 2)
    chol = position_std_dev_cholesky.reshape(B, Q, 2, 2)
    packed = jnp.concatenate(
        [pos, chol[..., 0, 0:1], chol[..., 1, 0:1], chol[..., 1, 1:2],
         jnp.zeros((B, Q, 3), jnp.float32)], axis=-1)
    occ_oh = (occupancy_target[:, None] ==
              jnp.arange(C_OCC, dtype=jnp.int32)[None, :]).astype(jnp.float32)

    k = jnp.minimum(jnp.arange(WPAD, dtype=jnp.int32), NWIN - 1)
    boff = (k // WIN) * (W * Q) + (k % WIN) * Q
    toff = (k // WIN) * (W * T) + (k % WIN) * T

    gbin, gtrue = _sc_gather_call(
        binary_mask_logits.reshape(-1), true_segmap.reshape(-1),
        inc_t.reshape(-1), matched_pred.reshape(-1), boff, toff)

    partial = pl.pallas_call(
        _main_kernel,
        grid=(B,),
        in_specs=[
            pl.BlockSpec((1, HW, Q), lambda b: (b, 0, 0)),
            pl.BlockSpec((1, HW, T), lambda b: (b, 0, 0)),
            pl.BlockSpec((1, 1, T), lambda b: (b, 0, 0)),
            pl.BlockSpec((1, 2, T), lambda b: (b, 0, 0)),
            pl.BlockSpec((1, 1, Q), lambda b: (b, 0, 0)),
            pl.BlockSpec((1, Q, 8), lambda b: (b, 0, 0)),
            pl.BlockSpec((B, C_OCC), lambda b: (0, 0)),
            pl.BlockSpec((B, C_OCC), lambda b: (0, 0)),
        ],
        out_specs=pl.BlockSpec(memory_space=pltpu.SMEM),
        out_shape=jax.ShapeDtypeStruct((1,), jnp.float32),
        scratch_shapes=[pltpu.SMEM((8,), jnp.float32)],
    )(portion, true, matched3, inc_t, ie, packed, occupancy_logits, occ_oh)

    out = pl.pallas_call(
        _combine_kernel,
        in_specs=[
            pl.BlockSpec((PAIRS // 2, 2 * WPAD), lambda: (0, 0)),
            pl.BlockSpec((PAIRS // 2, 2 * WPAD), lambda: (0, 0)),
            pl.BlockSpec(memory_space=pltpu.SMEM),
        ],
        out_specs=pl.BlockSpec(memory_space=pltpu.SMEM),
        out_shape=jax.ShapeDtypeStruct((1,), jnp.float32),
    )(gbin, gtrue, partial)
    return out[0]


# speedup vs baseline: 1.9712x; 1.9388x over previous
import jax
import jax.numpy as jnp
from jax import lax
from jax.experimental import pallas as pl
from jax.experimental.pallas import tpu as pltpu

B, Q, T, H, W = 4, 128, 64, 64, 64
HW = H * W
WIN = 7
NWIN = WIN * WIN
HALF = WIN // 2
C_OCC = 8
NO_ELECTRON_WEIGHT = 0.1
LOG_2PI = 1.8378770664093453


def _softplus(x):
    return jnp.maximum(x, 0.0) + jnp.log1p(jnp.exp(-jnp.abs(x)))


def _loss_kernel(portion_ref, binary_ref, true_ref, matched_ref, inc_ref,
                 ie_ref, pos_ref, chol_ref, occ_ref, occ_oh_ref, out_ref, acc_ref):
    b = pl.program_id(0)

    matched = matched_ref[0]
    q_iota = lax.broadcasted_iota(jnp.int32, (Q, T), 0)
    onehot = (q_iota == matched).astype(jnp.float32)

    true_b = true_ref[0]

    rp = lax.dot_general(
        portion_ref[0], onehot, (((1,), (0,)), ((), ())),
        precision=lax.Precision.DEFAULT,
        preferred_element_type=jnp.float32)
    e = jnp.exp(-jnp.abs(rp))
    p = jnp.where(rp >= 0.0, 1.0, e) / (1.0 + e)
    num_t = 2.0 * jnp.sum(p * true_b, axis=0, keepdims=True)
    den_t = jnp.sum(p, axis=0, keepdims=True) + jnp.sum(true_b, axis=0, keepdims=True)
    dice_b = jnp.sum(1.0 - (num_t + 1.0) / (den_t + 1.0))

    rb = lax.dot_general(
        binary_ref[0], onehot, (((1,), (0,)), ((), ())),
        precision=lax.Precision.DEFAULT,
        preferred_element_type=jnp.float32)
    r_t = jnp.floor(inc_ref[0, 0:1, :]).astype(jnp.int32)
    c_t = jnp.floor(inc_ref[0, 1:2, :]).astype(jnp.int32)
    pix = lax.broadcasted_iota(jnp.int32, (HW, T), 0)
    hh = pix // W
    ww = pix % W
    inwin = ((jnp.abs(hh - r_t) <= HALF) & (jnp.abs(ww - c_t) <= HALF))
    bce_el = _softplus(rb) - rb * true_b
    bce_b = jnp.sum(jnp.where(inwin, bce_el, 0.0))

    labels = jnp.max(onehot, axis=1, keepdims=True)
    wts = jnp.where(labels > 0.0, 1.0, NO_ELECTRON_WEIGHT)
    x_ie = ie_ref[0].reshape(Q, 1)
    class_b = jnp.sum(wts * (_softplus(x_ie) - x_ie * labels))

    packed = jnp.concatenate([pos_ref[0], chol_ref[0]], axis=1)
    g = lax.dot_general(
        onehot, packed, (((0,), (0,)), ((), ())),
        precision=lax.Precision.HIGHEST,
        preferred_element_type=jnp.float32)
    ix = inc_ref[0, 0:1, :].reshape(T, 1)
    iy = inc_ref[0, 1:2, :].reshape(T, 1)
    d0 = ix - g[:, 0:1]
    d1 = iy - g[:, 1:2]
    l00 = g[:, 2:3]
    l10 = g[:, 4:5]
    l11 = g[:, 5:6]
    z0 = d0 / l00
    z1 = (d1 - l10 * z0) / l11
    nll_b = jnp.sum(0.5 * (z0 * z0 + z1 * z1)
                    + jnp.log(jnp.abs(l00)) + jnp.log(jnp.abs(l11)) + LOG_2PI)

    @pl.when(b == 0)
    def _init():
        for i in range(4):
            acc_ref[i] = 0.0

    acc_ref[0] = acc_ref[0] + class_b
    acc_ref[1] = acc_ref[1] + bce_b
    acc_ref[2] = acc_ref[2] + dice_b
    acc_ref[3] = acc_ref[3] + nll_b

    @pl.when(b == B - 1)
    def _final():
        xo = occ_ref[:, :]
        m = jnp.max(xo, axis=1, keepdims=True)
        lse = m + jnp.log(jnp.sum(jnp.exp(xo - m), axis=1, keepdims=True))
        occ_loss = -jnp.sum(occ_oh_ref[:, :] * (xo - lse)) / B
        out_ref[0] = (acc_ref[0] / (B * Q)
                      + acc_ref[1] / (B * T * NWIN)
                      + acc_ref[2] / (B * T)
                      + acc_ref[3] / (B * T)
                      + occ_loss)


@jax.jit
def kernel(is_electron_logit, positions, position_std_dev_cholesky, true_segmap,
           binary_mask_logits, portion_logits, occupancy_logits, incidence_points,
           matched_pred, occupancy_target):
    portion = portion_logits.reshape(B, HW, Q)
    binary = binary_mask_logits.reshape(B, HW, Q)
    true = true_segmap.reshape(B, HW, T)
    matched3 = matched_pred.reshape(B, 1, T)
    inc_t = incidence_points.transpose(0, 2, 1)
    ie = is_electron_logit.reshape(B, 1, Q)
    pos = positions.reshape(B, Q, 2)
    chol = position_std_dev_cholesky.reshape(B, Q, 4)
    occ_oh = (occupancy_target[:, None] ==
              jnp.arange(C_OCC, dtype=jnp.int32)[None, :]).astype(jnp.float32)

    out = pl.pallas_call(
        _loss_kernel,
        grid=(B,),
        in_specs=[
            pl.BlockSpec((1, HW, Q), lambda b: (b, 0, 0)),
            pl.BlockSpec((1, HW, Q), lambda b: (b, 0, 0)),
            pl.BlockSpec((1, HW, T), lambda b: (b, 0, 0)),
            pl.BlockSpec((1, 1, T), lambda b: (b, 0, 0)),
            pl.BlockSpec((1, 2, T), lambda b: (b, 0, 0)),
            pl.BlockSpec((1, 1, Q), lambda b: (b, 0, 0)),
            pl.BlockSpec((1, Q, 2), lambda b: (b, 0, 0)),
            pl.BlockSpec((1, Q, 4), lambda b: (b, 0, 0)),
            pl.BlockSpec((B, C_OCC), lambda b: (0, 0)),
            pl.BlockSpec((B, C_OCC), lambda b: (0, 0)),
        ],
        out_specs=pl.BlockSpec(memory_space=pltpu.SMEM),
        out_shape=jax.ShapeDtypeStruct((1,), jnp.float32),
        scratch_shapes=[pltpu.SMEM((8,), jnp.float32)],
    )(portion, binary, true, matched3, inc_t, ie, pos, chol,
      occupancy_logits, occ_oh)
    return out[0]
